# baseline (speedup 1.0000x reference)
import functools

import jax
import jax.numpy as jnp
from jax import lax
from jax.experimental import pallas as pl
from jax.experimental.pallas import tpu as pltpu
from jax.experimental.pallas import tpu_sc as plsc

_B, _S, _D = 4096, 50, 128
_SP = 56
_NW = 32
_BPW = _B // _NW
_IDX_W = _BPW * _SP

_mesh = plsc.VectorSubcoreMesh(core_axis_name="c", subcore_axis_name="s")


@functools.partial(
    pl.kernel,
    mesh=_mesh,
    out_type=jax.ShapeDtypeStruct((_B, _S, _D), jnp.float32),
    scratch_types=[
        pltpu.VMEM((_IDX_W,), jnp.int32),
        pltpu.VMEM((2, _SP, _D), jnp.float32),
        pltpu.SemaphoreType.DMA,
        pltpu.SemaphoreType.DMA,
    ],
    compiler_params=pltpu.CompilerParams(use_tc_tiling_on_sc=True),
)
def _gather_kernel(ids_hbm, table_hbm, out_hbm, idx_v, rows_v, gsem, ssem):
    wid = lax.axis_index("s") * 2 + lax.axis_index("c")
    base_b = wid * _BPW
    pltpu.sync_copy(ids_hbm.at[pl.ds(wid * _IDX_W, _IDX_W)], idx_v)

    def gather(c, buf):
        pltpu.async_copy(
            table_hbm.at[idx_v.at[pl.ds(c * _SP, _SP)]], rows_v.at[buf], gsem)

    def gwait(buf):
        pltpu.make_async_copy(
            table_hbm.at[pl.ds(0, _SP)], rows_v.at[buf], gsem).wait()

    def scatter(c, buf):
        pltpu.async_copy(
            rows_v.at[buf, pl.ds(0, _S)], out_hbm.at[base_b + c], ssem)

    def swait(buf):
        pltpu.make_async_copy(
            rows_v.at[buf, pl.ds(0, _S)], out_hbm.at[base_b], ssem).wait()

    gather(0, 0)
    gather(1, 1)

    def body(i, carry):
        c = 2 * i
        gwait(0)
        scatter(c, 0)
        swait(0)
        gather(c + 2, 0)
        gwait(1)
        scatter(c + 1, 1)
        swait(1)
        gather(c + 3, 1)
        return carry

    lax.fori_loop(0, (_BPW - 2) // 2, body, 0)

    gwait(0)
    scatter(_BPW - 2, 0)
    gwait(1)
    scatter(_BPW - 1, 1)
    swait(0)
    swait(1)


def kernel(word_ids, table):
    ids = word_ids.astype(jnp.int32)
    ids_pad = jnp.pad(ids, ((0, 0), (0, _SP - _S))).reshape(-1)
    return _gather_kernel(ids_pad, table)


# speedup vs baseline: 1.0087x; 1.0087x over previous
import functools

import jax
import jax.numpy as jnp
from jax import lax
from jax.experimental import pallas as pl
from jax.experimental.pallas import tpu as pltpu
from jax.experimental.pallas import tpu_sc as plsc

_B, _S, _D = 4096, 50, 128
_SP = 56
_NW = 32
_BPW = _B // _NW
_IDX_W = _BPW * _SP
_NBB = 8
_CROWS = _NBB * _SP
_GCH = _BPW // _NBB

_mesh = plsc.VectorSubcoreMesh(core_axis_name="c", subcore_axis_name="s")


@functools.partial(
    pl.kernel,
    mesh=_mesh,
    out_type=jax.ShapeDtypeStruct((_B, _S, _D), jnp.float32),
    scratch_types=[
        pltpu.VMEM((_IDX_W,), jnp.int32),
        pltpu.VMEM((2, _CROWS, _D), jnp.float32),
        pltpu.SemaphoreType.DMA,
        pltpu.SemaphoreType.DMA,
    ],
    compiler_params=pltpu.CompilerParams(use_tc_tiling_on_sc=True),
)
def _gather_kernel(ids_hbm, table_hbm, out_hbm, idx_v, rows_v, gsem, ssem):
    wid = lax.axis_index("s") * 2 + lax.axis_index("c")
    base_b = wid * _BPW
    pltpu.sync_copy(ids_hbm.at[pl.ds(wid * _IDX_W, _IDX_W)], idx_v)

    def gather(c, buf):
        pltpu.async_copy(
            table_hbm.at[idx_v.at[pl.ds(c * _CROWS, _CROWS)]],
            rows_v.at[buf], gsem)

    def gwait(buf):
        pltpu.make_async_copy(
            table_hbm.at[pl.ds(0, _CROWS)], rows_v.at[buf], gsem).wait()

    def scatter(c, buf):
        for j in range(_NBB):
            pltpu.async_copy(
                rows_v.at[buf, pl.ds(j * _SP, _S)],
                out_hbm.at[base_b + c * _NBB + j], ssem)

    def swait(buf):
        for _ in range(_NBB):
            pltpu.make_async_copy(
                rows_v.at[buf, pl.ds(0, _S)], out_hbm.at[base_b], ssem).wait()

    gather(0, 0)
    gather(1, 1)

    def body(i, carry):
        c = 2 * i
        gwait(0)
        scatter(c, 0)
        swait(0)
        gather(c + 2, 0)
        gwait(1)
        scatter(c + 1, 1)
        swait(1)
        gather(c + 3, 1)
        return carry

    lax.fori_loop(0, (_GCH - 2) // 2, body, 0)

    gwait(0)
    scatter(_GCH - 2, 0)
    gwait(1)
    scatter(_GCH - 1, 1)
    swait(0)
    swait(1)


def kernel(word_ids, table):
    ids = word_ids.astype(jnp.int32)
    ids_pad = jnp.pad(ids, ((0, 0), (0, _SP - _S))).reshape(-1)
    return _gather_kernel(ids_pad, table)
